# Initial kernel scaffold; baseline (speedup 1.0000x reference)
import jax
import jax.numpy as jnp
from jax.experimental import pallas as pl


def kernel(x, edge_index, edge_attr, W_lin, b_lin, W_up, b_up):
    raise NotImplementedError("write your pallas kernel here")


# speedup vs baseline: 3.4559x; 3.4559x over previous
import functools

import jax
import jax.numpy as jnp
from jax import lax
from jax.experimental import pallas as pl
from jax.experimental.pallas import tpu as pltpu
from jax.experimental.pallas import tpu_sc as plsc

N, E, D_IN, D_EDGE, D_OUT = 10000, 320000, 128, 16, 128
NC, NS = 2, 16
NW = NC * NS
EPW = E // NW
CH = 80
NCHUNK = EPW // CH
NPAD = 10240
RPT = NPAD // NS
ZR = 80


def _pre_node_body(x_ref, w1_ref, w2_ref, wup_ref, bup_ref, a_ref, b_ref,
                   up_ref):
    xb = x_ref[...]
    a_ref[...] = jnp.dot(xb, w1_ref[...], preferred_element_type=jnp.float32)
    b_ref[...] = jnp.dot(xb, w2_ref[...], preferred_element_type=jnp.float32)
    up_ref[...] = (
        jnp.dot(xb, wup_ref[...], preferred_element_type=jnp.float32)
        + bup_ref[...]
    )


def _pre_edge_body(ea_ref, w3_ref, bl_ref, c_ref):
    c_ref[...] = (
        jnp.dot(ea_ref[...], w3_ref[...], preferred_element_type=jnp.float32)
        + bl_ref[...]
    )


def _post_body(p0_ref, p1_ref, up_ref, o_ref):
    o_ref[...] = p0_ref[0] + p1_ref[0] + up_ref[...]


def _tc_pre_node(x, w1, w2, wup, bup):
    bn = 1000
    full = lambda i: (0, 0)
    return pl.pallas_call(
        _pre_node_body,
        grid=(N // bn,),
        in_specs=[
            pl.BlockSpec((bn, D_IN), lambda i: (i, 0)),
            pl.BlockSpec((D_IN, D_OUT), full),
            pl.BlockSpec((D_IN, D_OUT), full),
            pl.BlockSpec((D_IN, D_OUT), full),
            pl.BlockSpec((1, D_OUT), full),
        ],
        out_specs=[pl.BlockSpec((bn, D_OUT), lambda i: (i, 0))] * 3,
        out_shape=[jax.ShapeDtypeStruct((N, D_OUT), jnp.float32)] * 3,
    )(x, w1, w2, wup, bup.reshape(1, D_OUT))


def _tc_pre_edge(edge_attr, w3, bl):
    be = 2000
    return pl.pallas_call(
        _pre_edge_body,
        grid=(E // be,),
        in_specs=[
            pl.BlockSpec((be, D_EDGE), lambda i: (i, 0)),
            pl.BlockSpec((D_EDGE, D_OUT), lambda i: (0, 0)),
            pl.BlockSpec((1, D_OUT), lambda i: (0, 0)),
        ],
        out_specs=pl.BlockSpec((be, D_OUT), lambda i: (i, 0)),
        out_shape=jax.ShapeDtypeStruct((E, D_OUT), jnp.float32),
    )(edge_attr, w3, bl.reshape(1, D_OUT))


def _tc_post(partials, up):
    bn = 1000
    return pl.pallas_call(
        _post_body,
        grid=(N // bn,),
        in_specs=[
            pl.BlockSpec((1, bn, D_OUT), lambda i: (0, i, 0)),
            pl.BlockSpec((1, bn, D_OUT), lambda i: (1, i, 0)),
            pl.BlockSpec((bn, D_OUT), lambda i: (i, 0)),
        ],
        out_specs=pl.BlockSpec((bn, D_OUT), lambda i: (i, 0)),
        out_shape=jax.ShapeDtypeStruct((N, D_OUT), jnp.float32),
    )(partials, partials, up)


def _sc_body(a_hbm, b_hbm, c_hbm, dst_hbm, src_hbm, out_hbm,
             agg_s, dst_v, src_v, a_v, b_v, c_v, z_v, sem_a, sem_b, sem_c):
    cid = lax.axis_index("c")
    sid = lax.axis_index("s")
    wid = sid * NC + cid
    rbase = sid * RPT

    def fill_row(i, _):
        for j in range(D_OUT // 16):
            z_v[i, pl.ds(j * 16, 16)] = jnp.zeros((16,), jnp.float32)
        return 0
    lax.fori_loop(0, ZR, fill_row, 0)

    def zdma(t, _):
        pltpu.sync_copy(z_v, agg_s.at[pl.ds(rbase + t * ZR, ZR)])
        return 0
    lax.fori_loop(0, RPT // ZR, zdma, 0)

    plsc.subcore_barrier()

    ebase = wid * EPW

    def chunk(k, _):
        eb = ebase + k * CH
        pltpu.sync_copy(dst_hbm.at[pl.ds(eb, CH)], dst_v)
        pltpu.sync_copy(src_hbm.at[pl.ds(eb, CH)], src_v)
        ga = pltpu.async_copy(a_hbm.at[dst_v], a_v, sem_a)
        gb = pltpu.async_copy(b_hbm.at[src_v], b_v, sem_b)
        gc = pltpu.async_copy(c_hbm.at[pl.ds(eb, CH)], c_v, sem_c)
        ga.wait()
        gb.wait()
        gc.wait()

        def row(i, _):
            for j in range(D_OUT // 16):
                sl = pl.ds(j * 16, 16)
                v = a_v[i, sl] + b_v[i, sl] + c_v[i, sl]
                c_v[i, sl] = jnp.maximum(v, v * 0.01)
            return 0
        lax.fori_loop(0, CH, row, 0)

        pltpu.sync_copy(c_v, agg_s.at[dst_v], add=True)
        return 0

    lax.fori_loop(0, NCHUNK, chunk, 0)

    plsc.subcore_barrier()

    pltpu.sync_copy(agg_s.at[pl.ds(rbase, RPT)],
                    out_hbm.at[cid, pl.ds(rbase, RPT)])


@functools.cache
def _sc_edge_kernel_fn():
    return pl.kernel(
        _sc_body,
        out_type=jax.ShapeDtypeStruct((NC, NPAD, D_OUT), jnp.float32),
        mesh=plsc.VectorSubcoreMesh(core_axis_name="c", subcore_axis_name="s",
                                    num_cores=NC, num_subcores=NS),
        scratch_types=[
            pltpu.VMEM_SHARED((NPAD, D_OUT), jnp.float32),
            pltpu.VMEM((CH,), jnp.int32),
            pltpu.VMEM((CH,), jnp.int32),
            pltpu.VMEM((CH, D_OUT), jnp.float32),
            pltpu.VMEM((CH, D_OUT), jnp.float32),
            pltpu.VMEM((CH, D_OUT), jnp.float32),
            pltpu.VMEM((ZR, D_OUT), jnp.float32),
            pltpu.SemaphoreType.DMA,
            pltpu.SemaphoreType.DMA,
            pltpu.SemaphoreType.DMA,
        ],
    )


def kernel(x, edge_index, edge_attr, W_lin, b_lin, W_up, b_up):
    w1 = W_lin[:D_IN]
    w2 = W_lin[D_IN:2 * D_IN]
    w3 = W_lin[2 * D_IN:]
    a, b, up = _tc_pre_node(x, w1, w2, W_up, b_up)
    c = _tc_pre_edge(edge_attr, w3, b_lin)
    dst = edge_index[1].astype(jnp.int32)
    src = edge_index[0].astype(jnp.int32)
    partials = _sc_edge_kernel_fn()(a, b, c, dst, src)
    return _tc_post(partials, up)
